# baseline (speedup 1.0000x reference)
import functools
import jax
import jax.numpy as jnp
from jax import lax
from jax.experimental import pallas as pl
from jax.experimental.pallas import tpu as pltpu
from jax.experimental.pallas import tpu_sc as plsc

_N = 10000
_E = 320000
_G = 16
_DIMS = [128, 64, 32, 16, 8, 128]

_NC = 2
_NS = 16
_NW = _NC * _NS

_CHUNK = 128
_EPW = 10240
_E_PAD = _NW * _EPW
_NCHUNK = _EPW // _CHUNK
_N_PAD = 10240
_RPW = _N_PAD // _NS
_BLK = 512
_NBLK = _N_PAD // _BLK

_mesh = plsc.VectorSubcoreMesh(core_axis_name="c", subcore_axis_name="s")


def _deg_body(dst_hbm, ones_hbm, zr_hbm, out_hbm, accsh, didx3, onesv, sem):
    c = lax.axis_index("c")
    s = lax.axis_index("s")
    w = s * _NC + c
    pltpu.sync_copy(zr_hbm, accsh.at[pl.ds(s * _RPW, _RPW)])
    pltpu.sync_copy(ones_hbm, onesv)
    pltpu.sync_copy(dst_hbm.at[w], didx3)
    plsc.subcore_barrier()

    def body(i, _):
        pltpu.sync_copy(onesv, accsh.at[didx3.at[i, 0]], add=True)
        return _

    lax.fori_loop(0, _NCHUNK, body, None)
    plsc.subcore_barrier()
    off2 = pl.multiple_of(c * _N_PAD + s * _RPW, _RPW)
    pltpu.sync_copy(accsh.at[pl.ds(s * _RPW, _RPW)], out_hbm.at[pl.ds(off2, _RPW)])


_sc_params = pltpu.CompilerParams(use_tc_tiling_on_sc=False)

_deg_kernel = functools.partial(
    pl.kernel,
    out_type=jax.ShapeDtypeStruct((_NC * _N_PAD, 8), jnp.float32),
    mesh=_mesh,
    compiler_params=_sc_params,
    scratch_types=[
        pltpu.VMEM_SHARED((_N_PAD, 8), jnp.float32),
        pltpu.VMEM((_NCHUNK, 1, _CHUNK), jnp.int32),
        pltpu.VMEM((_CHUNK, 8), jnp.float32),
        pltpu.SemaphoreType.DMA,
    ],
)(_deg_body)


def _make_agg(dout, chunk):
    nchunk = _EPW // chunk
    stage = dout <= 64

    def _ix(r, j):
        return r.at[j, 0]

    def _agg_body(gh_hbm, src_hbm, dst_hbm, zr_hbm, out_hbm, accsh, sidx3,
                  didx3, rows2, gsems, ssems, *maybe_ghs):
        c = lax.axis_index("c")
        s = lax.axis_index("s")
        w = s * _NC + c
        d0 = pltpu.async_copy(zr_hbm, accsh.at[pl.ds(s * _RPW, _RPW)],
                              gsems.at[0])
        d1 = pltpu.async_copy(src_hbm.at[w], sidx3, gsems.at[1])
        d2 = pltpu.async_copy(dst_hbm.at[w], didx3, gsems.at[2])
        if stage:
            ghs = maybe_ghs[0]
            d3 = pltpu.async_copy(gh_hbm.at[pl.ds(s * _RPW, _RPW)],
                                  ghs.at[pl.ds(s * _RPW, _RPW)], ssems.at[0])
            d3.wait()
            gh_src = ghs
        else:
            gh_src = gh_hbm
        d0.wait()
        d1.wait()
        d2.wait()
        plsc.subcore_barrier()

        pltpu.async_copy(gh_src.at[_ix(sidx3, 0)], rows2.at[0], gsems.at[0])
        pltpu.async_copy(gh_src.at[_ix(sidx3, 1)], rows2.at[1], gsems.at[1])

        def body(j, _):
            p = lax.rem(j, 3)

            @pl.when(j >= 1)
            def _():
                q = lax.rem(j - 1, 3)
                pltpu.make_async_copy(rows2.at[q],
                                      accsh.at[_ix(didx3, j - 1)],
                                      ssems.at[q]).wait()

            @pl.when(j + 2 < nchunk)
            def _():
                q = lax.rem(j + 2, 3)
                pltpu.async_copy(gh_src.at[_ix(sidx3, j + 2)], rows2.at[q],
                                 gsems.at[q])

            pltpu.make_async_copy(gh_src.at[_ix(sidx3, j)], rows2.at[p],
                                  gsems.at[p]).wait()
            pltpu.async_copy(rows2.at[p], accsh.at[_ix(didx3, j)],
                             ssems.at[p], add=True)
            return _

        lax.fori_loop(0, nchunk, body, None)
        q = (nchunk - 1) % 3
        pltpu.make_async_copy(rows2.at[q], accsh.at[_ix(didx3, nchunk - 1)],
                              ssems.at[q]).wait()
        plsc.subcore_barrier()
        off2 = pl.multiple_of(c * _N_PAD + s * _RPW, _RPW)
        pltpu.sync_copy(accsh.at[pl.ds(s * _RPW, _RPW)],
                        out_hbm.at[pl.ds(off2, _RPW)])

    return functools.partial(
        pl.kernel,
        out_type=jax.ShapeDtypeStruct((_NC * _N_PAD, dout), jnp.float32),
        mesh=_mesh,
        compiler_params=_sc_params,
        scratch_types=[
            pltpu.VMEM_SHARED((_N_PAD, dout), jnp.float32),
            pltpu.VMEM((nchunk, 1, chunk), jnp.int32),
            pltpu.VMEM((nchunk, 1, chunk), jnp.int32),
            pltpu.VMEM((3, chunk, dout), jnp.float32),
            pltpu.SemaphoreType.DMA((3,)),
            pltpu.SemaphoreType.DMA((3,)),
        ] + ([pltpu.VMEM_SHARED((_N_PAD, dout), jnp.float32)] if stage else []),
    )(_agg_body)


_agg_chunk = {64: 128, 32: 256, 16: 256, 8: 256}
_agg_kernels = {d: _make_agg(d, _agg_chunk[d]) for d in _agg_chunk}


def _tcg1_body(x_ref, w_ref, out_ref):
    out_ref[...] = jnp.dot(x_ref[...], w_ref[...],
                           preferred_element_type=jnp.float32)


def _tcg1(x_pad, w1):
    d1 = _DIMS[1]
    return pl.pallas_call(
        _tcg1_body,
        grid=(_NBLK,),
        in_specs=[
            pl.BlockSpec((_BLK, _DIMS[0]), lambda i: (i, 0)),
            pl.BlockSpec((_DIMS[0], d1), lambda i: (0, 0)),
        ],
        out_specs=pl.BlockSpec((_BLK, d1), lambda i: (i, 0)),
        out_shape=jax.ShapeDtypeStruct((_N_PAD, d1), jnp.float32),
    )(x_pad, w1)


def _tc0b_body(degp_ref, g_ref, gh_ref, dinv_ref):
    deg = degp_ref[0, :, 0:1] + degp_ref[1, :, 0:1] + 1.0
    dinv = lax.rsqrt(deg)
    dinv_ref[...] = dinv
    gh_ref[...] = dinv * g_ref[...]


def _tc0b(degp, g1):
    d1 = _DIMS[1]
    return pl.pallas_call(
        _tc0b_body,
        grid=(_NBLK,),
        in_specs=[
            pl.BlockSpec((2, _BLK, 8), lambda i: (0, i, 0)),
            pl.BlockSpec((_BLK, d1), lambda i: (i, 0)),
        ],
        out_specs=[
            pl.BlockSpec((_BLK, d1), lambda i: (i, 0)),
            pl.BlockSpec((_BLK, 1), lambda i: (i, 0)),
        ],
        out_shape=[
            jax.ShapeDtypeStruct((_N_PAD, d1), jnp.float32),
            jax.ShapeDtypeStruct((_N_PAD, 1), jnp.float32),
        ],
    )(degp.reshape(2, _N_PAD, 8), g1)


def _tcmid_body(a0_ref, a1_ref, gh_ref, dinv_ref, b_ref, w_ref, out_ref):
    i = pl.program_id(0)
    dinv = dinv_ref[...]
    h = jnp.maximum(dinv * (a0_ref[...] + a1_ref[...] + gh_ref[...]) + b_ref[...], 0.0)
    rowid = i * _BLK + lax.broadcasted_iota(jnp.int32, (_BLK, 1), 0)
    h = jnp.where(rowid < _N, h, 0.0)
    out_ref[...] = dinv * jnp.dot(h, w_ref[...], preferred_element_type=jnp.float32)


def _tcmid(accflat, gh, dinv, b2d, wnext, din, dnext):
    return pl.pallas_call(
        _tcmid_body,
        grid=(_NBLK,),
        in_specs=[
            pl.BlockSpec((_BLK, din), lambda i: (i, 0)),
            pl.BlockSpec((_BLK, din), lambda i: (i + _NBLK, 0)),
            pl.BlockSpec((_BLK, din), lambda i: (i, 0)),
            pl.BlockSpec((_BLK, 1), lambda i: (i, 0)),
            pl.BlockSpec((1, din), lambda i: (0, 0)),
            pl.BlockSpec((din, dnext), lambda i: (0, 0)),
        ],
        out_specs=pl.BlockSpec((_BLK, dnext), lambda i: (i, 0)),
        out_shape=jax.ShapeDtypeStruct((_N_PAD, dnext), jnp.float32),
    )(accflat, accflat, gh, dinv, b2d, wnext)


def _tcmid_nomm_body(a0_ref, a1_ref, gh_ref, dinv_ref, b_ref, out_ref):
    i = pl.program_id(0)
    dinv = dinv_ref[...]
    h = jnp.maximum(dinv * (a0_ref[...] + a1_ref[...] + gh_ref[...]) + b_ref[...], 0.0)
    rowid = i * _BLK + lax.broadcasted_iota(jnp.int32, (_BLK, 1), 0)
    h = jnp.where(rowid < _N, h, 0.0)
    out_ref[...] = dinv * h


def _tcmid_nomm(accflat, gh, dinv, b2d, din):
    return pl.pallas_call(
        _tcmid_nomm_body,
        grid=(_NBLK,),
        in_specs=[
            pl.BlockSpec((_BLK, din), lambda i: (i, 0)),
            pl.BlockSpec((_BLK, din), lambda i: (i + _NBLK, 0)),
            pl.BlockSpec((_BLK, din), lambda i: (i, 0)),
            pl.BlockSpec((_BLK, 1), lambda i: (i, 0)),
            pl.BlockSpec((1, din), lambda i: (0, 0)),
        ],
        out_specs=pl.BlockSpec((_BLK, din), lambda i: (i, 0)),
        out_shape=jax.ShapeDtypeStruct((_N_PAD, din), jnp.float32),
    )(accflat, accflat, gh, dinv, b2d)


def _tc5_body(a0_ref, a1_ref, ghp_ref, dinv_ref, w_ref, b_ref, batch_ref,
              out_ref, sums_scr, cnt_scr):
    i = pl.program_id(0)

    @pl.when(i == 0)
    def _():
        sums_scr[...] = jnp.zeros_like(sums_scr)
        cnt_scr[...] = jnp.zeros_like(cnt_scr)

    dinv = dinv_ref[...]
    t = dinv * (a0_ref[...] + a1_ref[...] + ghp_ref[...])
    h = jnp.maximum(jnp.dot(t, w_ref[...], preferred_element_type=jnp.float32)
                    + b_ref[...], 0.0)
    rowid = i * _BLK + lax.broadcasted_iota(jnp.int32, (_BLK, 1), 0)
    h = jnp.where(rowid < _N, h, 0.0)
    gids = lax.broadcasted_iota(jnp.int32, (_BLK, _G), 1)
    mask = (batch_ref[...] == gids).astype(jnp.float32)
    dn = (((0,), (0,)), ((), ()))
    sums_scr[...] += lax.dot_general(mask, h, dn, preferred_element_type=jnp.float32)
    ones = jnp.ones((_BLK, 1), jnp.float32)
    cnt_scr[...] += lax.dot_general(mask, ones, dn, preferred_element_type=jnp.float32)

    @pl.when(i == _NBLK - 1)
    def _():
        out_ref[...] = sums_scr[...] / jnp.maximum(cnt_scr[...], 1.0)


def _tc5(accflat, ghp, dinv, w5, b2d, batch2d):
    din, d5 = _DIMS[4], _DIMS[5]
    return pl.pallas_call(
        _tc5_body,
        grid=(_NBLK,),
        in_specs=[
            pl.BlockSpec((_BLK, din), lambda i: (i, 0)),
            pl.BlockSpec((_BLK, din), lambda i: (i + _NBLK, 0)),
            pl.BlockSpec((_BLK, din), lambda i: (i, 0)),
            pl.BlockSpec((_BLK, 1), lambda i: (i, 0)),
            pl.BlockSpec((din, d5), lambda i: (0, 0)),
            pl.BlockSpec((1, d5), lambda i: (0, 0)),
            pl.BlockSpec((_BLK, 1), lambda i: (i, 0)),
        ],
        out_specs=pl.BlockSpec((_G, d5), lambda i: (0, 0)),
        out_shape=jax.ShapeDtypeStruct((_G, d5), jnp.float32),
        scratch_shapes=[
            pltpu.VMEM((_G, d5), jnp.float32),
            pltpu.VMEM((_G, 1), jnp.float32),
        ],
    )(accflat, accflat, ghp, dinv, w5, b2d, batch2d)


def kernel(x, edge_index, batch, W1, b1, W2, b2, W3, b3, W4, b4, W5, b5):
    src = edge_index[0].astype(jnp.int32)
    dst = edge_index[1].astype(jnp.int32)
    pad_e = jnp.full((_E_PAD - _E,), _N, jnp.int32)
    src_f = jnp.concatenate([src, pad_e])
    dst_f = jnp.concatenate([dst, pad_e])
    dst_p = dst_f.reshape(_NW, _NCHUNK, 1, _CHUNK)

    def _eidx(flat, chunk):
        return flat.reshape(_NW, _EPW // chunk, 1, chunk)

    x_pad = jnp.zeros((_N_PAD, _DIMS[0]), jnp.float32).at[:_N].set(x)
    batch2d = jnp.full((_N_PAD, 1), _G, jnp.int32).at[:_N, 0].set(batch.astype(jnp.int32))

    ones8 = jnp.ones((_CHUNK, 8), jnp.float32)
    zr8 = jnp.zeros((_RPW, 8), jnp.float32)

    g1 = _tcg1(x_pad, W1)
    degp = _deg_kernel(dst_p, ones8, zr8)
    gh, dinv = _tc0b(degp, g1)

    params = [(W2, b1), (W3, b2), (W4, b3)]
    for l in range(1, 4):
        din, dnext = _DIMS[l], _DIMS[l + 1]
        ch = _agg_chunk[din]
        zr = jnp.zeros((_RPW, din), jnp.float32)
        accflat = _agg_kernels[din](gh, _eidx(src_f, ch), _eidx(dst_f, ch), zr)
        wnext, b = params[l - 1]
        gh = _tcmid(accflat, gh, dinv, b.reshape(1, din), wnext, din, dnext)

    din = _DIMS[4]
    ch = _agg_chunk[din]
    zr = jnp.zeros((_RPW, din), jnp.float32)
    accflat = _agg_kernels[din](gh, _eidx(src_f, ch), _eidx(dst_f, ch), zr)
    ghp = _tcmid_nomm(accflat, gh, dinv, b4.reshape(1, din), din)

    accflat = _agg_kernels[din](ghp, _eidx(src_f, ch), _eidx(dst_f, ch), zr)
    return _tc5(accflat, ghp, dinv, W5, b5.reshape(1, _DIMS[5]), batch2d)


# speedup vs baseline: 1.0021x; 1.0021x over previous
import functools
import jax
import jax.numpy as jnp
from jax import lax
from jax.experimental import pallas as pl
from jax.experimental.pallas import tpu as pltpu
from jax.experimental.pallas import tpu_sc as plsc

_N = 10000
_E = 320000
_G = 16
_DIMS = [128, 64, 32, 16, 8, 128]

_NC = 2
_NS = 16
_NW = _NC * _NS

_CHUNK = 128
_EPW = 10240
_E_PAD = _NW * _EPW
_NCHUNK = _EPW // _CHUNK
_N_PAD = 10240
_RPW = _N_PAD // _NS
_BLK = 512
_NBLK = _N_PAD // _BLK

_mesh = plsc.VectorSubcoreMesh(core_axis_name="c", subcore_axis_name="s")


def _deg_body(dst_hbm, ones_hbm, zr_hbm, out_hbm, accsh, didx3, onesv, sem):
    c = lax.axis_index("c")
    s = lax.axis_index("s")
    w = s * _NC + c
    pltpu.sync_copy(zr_hbm, accsh.at[pl.ds(s * _RPW, _RPW)])
    pltpu.sync_copy(ones_hbm, onesv)
    pltpu.sync_copy(dst_hbm.at[w], didx3)
    plsc.subcore_barrier()

    def body(i, _):
        pltpu.sync_copy(onesv, accsh.at[didx3.at[i, 0]], add=True)
        return _

    lax.fori_loop(0, _NCHUNK, body, None)
    plsc.subcore_barrier()
    off2 = pl.multiple_of(c * _N_PAD + s * _RPW, _RPW)
    pltpu.sync_copy(accsh.at[pl.ds(s * _RPW, _RPW)], out_hbm.at[pl.ds(off2, _RPW)])


_sc_params = pltpu.CompilerParams(use_tc_tiling_on_sc=False)

_deg_kernel = functools.partial(
    pl.kernel,
    out_type=jax.ShapeDtypeStruct((_NC * _N_PAD, 8), jnp.float32),
    mesh=_mesh,
    compiler_params=_sc_params,
    scratch_types=[
        pltpu.VMEM_SHARED((_N_PAD, 8), jnp.float32),
        pltpu.VMEM((_NCHUNK, 1, _CHUNK), jnp.int32),
        pltpu.VMEM((_CHUNK, 8), jnp.float32),
        pltpu.SemaphoreType.DMA,
    ],
)(_deg_body)


def _make_agg(dout, chunk):
    nchunk = _EPW // chunk
    stage = dout <= 64

    def _ix(r, j):
        return r.at[j, 0]

    def _agg_body(gh_hbm, src_hbm, dst_hbm, zr_hbm, out_hbm, accsh, sidx3,
                  didx3, rows2, gsems, ssems, *maybe_ghs):
        c = lax.axis_index("c")
        s = lax.axis_index("s")
        w = s * _NC + c
        d0 = pltpu.async_copy(zr_hbm, accsh.at[pl.ds(s * _RPW, _RPW)],
                              gsems.at[0])
        d1 = pltpu.async_copy(src_hbm.at[w], sidx3, gsems.at[1])
        d2 = pltpu.async_copy(dst_hbm.at[w], didx3, gsems.at[2])
        if stage:
            ghs = maybe_ghs[0]
            d3 = pltpu.async_copy(gh_hbm.at[pl.ds(s * _RPW, _RPW)],
                                  ghs.at[pl.ds(s * _RPW, _RPW)], ssems.at[0])
            d3.wait()
            gh_src = ghs
        else:
            gh_src = gh_hbm
        d0.wait()
        d1.wait()
        d2.wait()
        plsc.subcore_barrier()

        pltpu.async_copy(gh_src.at[_ix(sidx3, 0)], rows2.at[0], gsems.at[0])
        pltpu.async_copy(gh_src.at[_ix(sidx3, 1)], rows2.at[1], gsems.at[1])

        def body(j, _):
            p = lax.rem(j, 3)

            @pl.when(j >= 1)
            def _():
                q = lax.rem(j - 1, 3)
                pltpu.make_async_copy(rows2.at[q],
                                      accsh.at[_ix(didx3, j - 1)],
                                      ssems.at[q]).wait()

            @pl.when(j + 2 < nchunk)
            def _():
                q = lax.rem(j + 2, 3)
                pltpu.async_copy(gh_src.at[_ix(sidx3, j + 2)], rows2.at[q],
                                 gsems.at[q])

            pltpu.make_async_copy(gh_src.at[_ix(sidx3, j)], rows2.at[p],
                                  gsems.at[p]).wait()
            pltpu.async_copy(rows2.at[p], accsh.at[_ix(didx3, j)],
                             ssems.at[p], add=True)
            return _

        lax.fori_loop(0, nchunk, body, None)
        q = (nchunk - 1) % 3
        pltpu.make_async_copy(rows2.at[q], accsh.at[_ix(didx3, nchunk - 1)],
                              ssems.at[q]).wait()
        plsc.subcore_barrier()
        off2 = pl.multiple_of(c * _N_PAD + s * _RPW, _RPW)
        pltpu.sync_copy(accsh.at[pl.ds(s * _RPW, _RPW)],
                        out_hbm.at[pl.ds(off2, _RPW)])

    return functools.partial(
        pl.kernel,
        out_type=jax.ShapeDtypeStruct((_NC * _N_PAD, dout), jnp.float32),
        mesh=_mesh,
        compiler_params=_sc_params,
        scratch_types=[
            pltpu.VMEM_SHARED((_N_PAD, dout), jnp.float32),
            pltpu.VMEM((nchunk, 1, chunk), jnp.int32),
            pltpu.VMEM((nchunk, 1, chunk), jnp.int32),
            pltpu.VMEM((3, chunk, dout), jnp.float32),
            pltpu.SemaphoreType.DMA((3,)),
            pltpu.SemaphoreType.DMA((3,)),
        ] + ([pltpu.VMEM_SHARED((_N_PAD, dout), jnp.float32)] if stage else []),
    )(_agg_body)


_agg_chunk = {64: 128, 32: 256, 16: 256, 8: 256}
_agg_kernels = {d: _make_agg(d, _agg_chunk[d]) for d in _agg_chunk}


def _tc0_body(degp_ref, x_ref, w_ref, gh_ref, dinv_ref):
    deg = degp_ref[0, :, 0:1] + degp_ref[1, :, 0:1] + 1.0
    dinv = lax.rsqrt(deg)
    dinv_ref[...] = dinv
    gh_ref[...] = dinv * jnp.dot(x_ref[...], w_ref[...],
                                 preferred_element_type=jnp.float32)


def _tc0(degp, x_pad, w1):
    d1 = _DIMS[1]
    return pl.pallas_call(
        _tc0_body,
        grid=(_NBLK,),
        in_specs=[
            pl.BlockSpec((2, _BLK, 8), lambda i: (0, i, 0)),
            pl.BlockSpec((_BLK, _DIMS[0]), lambda i: (i, 0)),
            pl.BlockSpec((_DIMS[0], d1), lambda i: (0, 0)),
        ],
        out_specs=[
            pl.BlockSpec((_BLK, d1), lambda i: (i, 0)),
            pl.BlockSpec((_BLK, 1), lambda i: (i, 0)),
        ],
        out_shape=[
            jax.ShapeDtypeStruct((_N_PAD, d1), jnp.float32),
            jax.ShapeDtypeStruct((_N_PAD, 1), jnp.float32),
        ],
    )(degp.reshape(2, _N_PAD, 8), x_pad, w1)


def _tcmid_body(a0_ref, a1_ref, gh_ref, dinv_ref, b_ref, w_ref, out_ref):
    i = pl.program_id(0)
    dinv = dinv_ref[...]
    h = jnp.maximum(dinv * (a0_ref[...] + a1_ref[...] + gh_ref[...]) + b_ref[...], 0.0)
    rowid = i * _BLK + lax.broadcasted_iota(jnp.int32, (_BLK, 1), 0)
    h = jnp.where(rowid < _N, h, 0.0)
    out_ref[...] = dinv * jnp.dot(h, w_ref[...], preferred_element_type=jnp.float32)


def _tcmid(accflat, gh, dinv, b2d, wnext, din, dnext):
    return pl.pallas_call(
        _tcmid_body,
        grid=(_NBLK,),
        in_specs=[
            pl.BlockSpec((_BLK, din), lambda i: (i, 0)),
            pl.BlockSpec((_BLK, din), lambda i: (i + _NBLK, 0)),
            pl.BlockSpec((_BLK, din), lambda i: (i, 0)),
            pl.BlockSpec((_BLK, 1), lambda i: (i, 0)),
            pl.BlockSpec((1, din), lambda i: (0, 0)),
            pl.BlockSpec((din, dnext), lambda i: (0, 0)),
        ],
        out_specs=pl.BlockSpec((_BLK, dnext), lambda i: (i, 0)),
        out_shape=jax.ShapeDtypeStruct((_N_PAD, dnext), jnp.float32),
    )(accflat, accflat, gh, dinv, b2d, wnext)


def _tcmid_nomm_body(a0_ref, a1_ref, gh_ref, dinv_ref, b_ref, out_ref):
    i = pl.program_id(0)
    dinv = dinv_ref[...]
    h = jnp.maximum(dinv * (a0_ref[...] + a1_ref[...] + gh_ref[...]) + b_ref[...], 0.0)
    rowid = i * _BLK + lax.broadcasted_iota(jnp.int32, (_BLK, 1), 0)
    h = jnp.where(rowid < _N, h, 0.0)
    out_ref[...] = dinv * h


def _tcmid_nomm(accflat, gh, dinv, b2d, din):
    return pl.pallas_call(
        _tcmid_nomm_body,
        grid=(_NBLK,),
        in_specs=[
            pl.BlockSpec((_BLK, din), lambda i: (i, 0)),
            pl.BlockSpec((_BLK, din), lambda i: (i + _NBLK, 0)),
            pl.BlockSpec((_BLK, din), lambda i: (i, 0)),
            pl.BlockSpec((_BLK, 1), lambda i: (i, 0)),
            pl.BlockSpec((1, din), lambda i: (0, 0)),
        ],
        out_specs=pl.BlockSpec((_BLK, din), lambda i: (i, 0)),
        out_shape=jax.ShapeDtypeStruct((_N_PAD, din), jnp.float32),
    )(accflat, accflat, gh, dinv, b2d)


def _tc5_body(a0_ref, a1_ref, ghp_ref, dinv_ref, w_ref, b_ref, batch_ref,
              out_ref, sums_scr, cnt_scr):
    i = pl.program_id(0)

    @pl.when(i == 0)
    def _():
        sums_scr[...] = jnp.zeros_like(sums_scr)
        cnt_scr[...] = jnp.zeros_like(cnt_scr)

    dinv = dinv_ref[...]
    t = dinv * (a0_ref[...] + a1_ref[...] + ghp_ref[...])
    h = jnp.maximum(jnp.dot(t, w_ref[...], preferred_element_type=jnp.float32)
                    + b_ref[...], 0.0)
    rowid = i * _BLK + lax.broadcasted_iota(jnp.int32, (_BLK, 1), 0)
    h = jnp.where(rowid < _N, h, 0.0)
    gids = lax.broadcasted_iota(jnp.int32, (_BLK, _G), 1)
    mask = (batch_ref[...] == gids).astype(jnp.float32)
    dn = (((0,), (0,)), ((), ()))
    sums_scr[...] += lax.dot_general(mask, h, dn, preferred_element_type=jnp.float32)
    ones = jnp.ones((_BLK, 1), jnp.float32)
    cnt_scr[...] += lax.dot_general(mask, ones, dn, preferred_element_type=jnp.float32)

    @pl.when(i == _NBLK - 1)
    def _():
        out_ref[...] = sums_scr[...] / jnp.maximum(cnt_scr[...], 1.0)


def _tc5(accflat, ghp, dinv, w5, b2d, batch2d):
    din, d5 = _DIMS[4], _DIMS[5]
    return pl.pallas_call(
        _tc5_body,
        grid=(_NBLK,),
        in_specs=[
            pl.BlockSpec((_BLK, din), lambda i: (i, 0)),
            pl.BlockSpec((_BLK, din), lambda i: (i + _NBLK, 0)),
            pl.BlockSpec((_BLK, din), lambda i: (i, 0)),
            pl.BlockSpec((_BLK, 1), lambda i: (i, 0)),
            pl.BlockSpec((din, d5), lambda i: (0, 0)),
            pl.BlockSpec((1, d5), lambda i: (0, 0)),
            pl.BlockSpec((_BLK, 1), lambda i: (i, 0)),
        ],
        out_specs=pl.BlockSpec((_G, d5), lambda i: (0, 0)),
        out_shape=jax.ShapeDtypeStruct((_G, d5), jnp.float32),
        scratch_shapes=[
            pltpu.VMEM((_G, d5), jnp.float32),
            pltpu.VMEM((_G, 1), jnp.float32),
        ],
    )(accflat, accflat, ghp, dinv, w5, b2d, batch2d)


def kernel(x, edge_index, batch, W1, b1, W2, b2, W3, b3, W4, b4, W5, b5):
    src = edge_index[0].astype(jnp.int32)
    dst = edge_index[1].astype(jnp.int32)
    pad_e = jnp.full((_E_PAD - _E,), _N, jnp.int32)
    src_f = jnp.concatenate([src, pad_e])
    dst_f = jnp.concatenate([dst, pad_e])
    dst_p = dst_f.reshape(_NW, _NCHUNK, 1, _CHUNK)

    def _eidx(flat, chunk):
        return flat.reshape(_NW, _EPW // chunk, 1, chunk)

    x_pad = jnp.zeros((_N_PAD, _DIMS[0]), jnp.float32).at[:_N].set(x)
    batch2d = jnp.full((_N_PAD, 1), _G, jnp.int32).at[:_N, 0].set(batch.astype(jnp.int32))

    ones8 = jnp.ones((_CHUNK, 8), jnp.float32)
    zr8 = jnp.zeros((_RPW, 8), jnp.float32)

    degp = _deg_kernel(dst_p, ones8, zr8)
    gh, dinv = _tc0(degp, x_pad, W1)

    params = [(W2, b1), (W3, b2), (W4, b3)]
    for l in range(1, 4):
        din, dnext = _DIMS[l], _DIMS[l + 1]
        ch = _agg_chunk[din]
        zr = jnp.zeros((_RPW, din), jnp.float32)
        accflat = _agg_kernels[din](gh, _eidx(src_f, ch), _eidx(dst_f, ch), zr)
        wnext, b = params[l - 1]
        gh = _tcmid(accflat, gh, dinv, b.reshape(1, din), wnext, din, dnext)

    din = _DIMS[4]
    ch = _agg_chunk[din]
    zr = jnp.zeros((_RPW, din), jnp.float32)
    accflat = _agg_kernels[din](gh, _eidx(src_f, ch), _eidx(dst_f, ch), zr)
    ghp = _tcmid_nomm(accflat, gh, dinv, b4.reshape(1, din), din)

    accflat = _agg_kernels[din](ghp, _eidx(src_f, ch), _eidx(dst_f, ch), zr)
    return _tc5(accflat, ghp, dinv, W5, b5.reshape(1, _DIMS[5]), batch2d)
